# baseline (speedup 1.0000x reference)
import functools

import jax
import jax.numpy as jnp
from jax import lax
from jax.experimental import pallas as pl
from jax.experimental.pallas import tpu as pltpu
from jax.experimental.pallas import tpu_sc as plsc

HIDDEN = 64
HALF = 32
CHUNK = 128
BN = 8192
QR = BN // 4


def _pack_half(table_t, h):
    v = table_t.shape[1]
    grid = -(-v // BN)
    vp = grid * QR

    def pack_k(a_ref, o_ref):
        for j in range(4):
            o_ref[:, j * HALF : (j + 1) * HALF] = jnp.transpose(
                a_ref[:, j * QR : (j + 1) * QR], (1, 0)
            )

    return pl.pallas_call(
        pack_k,
        grid=(grid,),
        in_specs=[pl.BlockSpec((HALF, BN), lambda i: (h, i))],
        out_specs=pl.BlockSpec((QR, 128), lambda i: (i, 0)),
        out_shape=jax.ShapeDtypeStruct((vp, 128), jnp.float32),
    )(table_t)


def _remap_indices(idx):
    i_blk = idx // BN
    q = idx % BN
    return i_blk * BN + (q % QR) * 4 + q // QR


def _gather_rows(idx2d, dst2d, table):
    n_chunks, chunk = idx2d.shape
    total = n_chunks * chunk
    info = plsc.get_sparse_core_info()
    nw = info.num_cores * info.num_subcores
    chunks_per_w = n_chunks // nw

    mesh = plsc.VectorSubcoreMesh(core_axis_name="c", subcore_axis_name="s")

    grp = 4
    n_groups = chunks_per_w // grp

    @functools.partial(
        pl.kernel,
        mesh=mesh,
        out_type=jax.ShapeDtypeStruct((total, HALF), jnp.float32),
        scratch_types=[
            pltpu.VMEM((chunks_per_w, chunk), jnp.int32),
            pltpu.VMEM((chunks_per_w, chunk), jnp.int32),
            pltpu.VMEM((2 * grp, chunk, HALF), jnp.float32),
            pltpu.SemaphoreType.DMA,
            pltpu.SemaphoreType.DMA,
        ],
        compiler_params=pltpu.CompilerParams(use_tc_tiling_on_sc=False),
    )
    def gather_k(idx_hbm, dst_hbm, table_hbm, out_hbm, idx_v, dst_v, rows_v, gsem, osem):
        wid = lax.axis_index("s") * info.num_cores + lax.axis_index("c")
        cbase = wid * chunks_per_w
        pltpu.sync_copy(idx_hbm.at[pl.ds(cbase, chunks_per_w)], idx_v)
        pltpu.sync_copy(dst_hbm.at[pl.ds(cbase, chunks_per_w)], dst_v)

        def fire_group(g, sb):
            for i in range(grp):
                pltpu.async_copy(
                    table_hbm.at[idx_v.at[g * grp + i]], rows_v.at[sb + i], gsem
                )

        def drain(sem, n):
            for _ in range(n):
                pltpu.make_async_copy(
                    table_hbm.at[pl.ds(0, chunk)], rows_v.at[0], sem
                ).wait()

        fire_group(0, 0)

        def body(g, carry):
            sb = (g % 2) * grp
            nsb = grp - sb
            drain(gsem, grp)

            @pl.when(g >= 1)
            def _():
                drain(osem, grp)

            @pl.when(g + 1 < n_groups)
            def _():
                fire_group(g + 1, nsb)

            for i in range(grp):
                pltpu.async_copy(
                    rows_v.at[sb + i],
                    out_hbm.at[dst_v.at[g * grp + i]],
                    osem,
                )
            return carry

        lax.fori_loop(0, n_groups, body, 0)
        drain(osem, grp)

    return gather_k(idx2d, dst2d, table)


def _project_half(g2, wh, init, bsz, nquad):
    bm = 2048
    nb = bsz // bm
    bias_row = init.shape[0] == 1

    def mm_k(x_ref, w_ref, i_ref, o_ref):
        u = pl.program_id(1)

        @pl.when(u == 0)
        def _():
            o_ref[...] = jnp.broadcast_to(i_ref[...], o_ref.shape)

        o_ref[...] += jnp.dot(
            x_ref[...], w_ref[...], preferred_element_type=jnp.float32
        )

    return pl.pallas_call(
        mm_k,
        grid=(nb, nquad),
        in_specs=[
            pl.BlockSpec((bm, 4 * HALF), lambda i, u: (u * nb + i, 0)),
            pl.BlockSpec((4 * HALF, HIDDEN), lambda i, u: (u, 0)),
            pl.BlockSpec(
                (1 if bias_row else bm, HIDDEN),
                (lambda i, u: (0, 0)) if bias_row else (lambda i, u: (i, 0)),
            ),
        ],
        out_specs=pl.BlockSpec((bm, HIDDEN), lambda i, u: (i, 0)),
        out_shape=jax.ShapeDtypeStruct((bsz, HIDDEN), jnp.float32),
    )(g2, wh, init)


def kernel(int_vals, table, W, b):
    bsz, s = int_vals.shape
    nquad = s // 4
    table_t = table.T
    idx2d = _remap_indices(int_vals).reshape(-1, CHUNK)
    bb = jnp.arange(bsz, dtype=jnp.int32)[:, None]
    ss = jnp.arange(s, dtype=jnp.int32)[None, :]
    dst2d = (4 * ((ss // 4) * bsz + bb) + ss % 4).reshape(-1, CHUNK)
    w3 = W.reshape(s, HIDDEN, HIDDEN)

    out = b.reshape(1, HIDDEN)
    for h in range(2):
        packed = _pack_half(table_t, h)
        lin = packed.reshape(4 * packed.shape[0], HALF)
        g = _gather_rows(idx2d, dst2d, lin)
        g2 = g.reshape(nquad * bsz, 4 * HALF)
        wh = w3[:, h * HALF : (h + 1) * HALF, :].reshape(s * HALF, HIDDEN)
        out = _project_half(g2, wh, out, bsz, nquad)
    return out


# speedup vs baseline: 1.4495x; 1.4495x over previous
import functools

import jax
import jax.numpy as jnp
from jax import lax
from jax.experimental import pallas as pl
from jax.experimental.pallas import tpu as pltpu
from jax.experimental.pallas import tpu_sc as plsc

HIDDEN = 64
CHUNK = 128
BN = 8192
BNP = BN // 2


def _pack_table(table_t):
    v = table_t.shape[1]
    grid = -(-v // BN)
    vp = grid * BNP

    sub = 1024

    def pack_k(a_ref, e_ref, o_ref):
        for q in range(BNP // sub):
            c = q * sub
            o_ref[q * sub : (q + 1) * sub, 0:64] = jnp.transpose(
                a_ref[:, c : c + sub], (1, 0)
            )
            c2 = BNP + q * sub
            o_ref[q * sub : (q + 1) * sub, 64:128] = lax.dot_general(
                a_ref[:, c2 : c2 + sub],
                e_ref[...],
                (((0,), (0,)), ((), ())),
                preferred_element_type=jnp.float32,
                precision=lax.Precision.DEFAULT,
            )

    return pl.pallas_call(
        pack_k,
        grid=(grid,),
        in_specs=[
            pl.BlockSpec((64, BN), lambda i: (0, i)),
            pl.BlockSpec((64, 64), lambda i: (0, 0)),
        ],
        out_specs=pl.BlockSpec((BNP, 128), lambda i: (i, 0)),
        out_shape=jax.ShapeDtypeStruct((vp, 128), jnp.float32),
    )(table_t, jnp.eye(64, dtype=jnp.float32))


def _remap_indices(idx):
    i_blk = idx // BN
    q = idx % BN
    return i_blk * BN + jnp.where(q < BNP, 2 * q, 2 * (q - BNP) + 1)


def _gather_rows(idx2d, dst2d, table):
    n_chunks, chunk = idx2d.shape
    total = n_chunks * chunk
    info = plsc.get_sparse_core_info()
    nw = info.num_cores * info.num_subcores
    chunks_per_w = n_chunks // nw

    mesh = plsc.VectorSubcoreMesh(core_axis_name="c", subcore_axis_name="s")

    grp = 4
    n_groups = chunks_per_w // grp

    @functools.partial(
        pl.kernel,
        mesh=mesh,
        out_type=jax.ShapeDtypeStruct((total, HIDDEN), jnp.float32),
        scratch_types=[
            pltpu.VMEM((chunks_per_w, chunk), jnp.int32),
            pltpu.VMEM((chunks_per_w, chunk), jnp.int32),
            pltpu.VMEM((2 * grp, chunk, HIDDEN), jnp.float32),
            pltpu.SemaphoreType.DMA,
            pltpu.SemaphoreType.DMA,
        ],
        compiler_params=pltpu.CompilerParams(use_tc_tiling_on_sc=False),
    )
    def gather_k(idx_hbm, dst_hbm, table_hbm, out_hbm, idx_v, dst_v, rows_v, gsem, osem):
        wid = lax.axis_index("s") * info.num_cores + lax.axis_index("c")
        cbase = wid * chunks_per_w
        pltpu.sync_copy(idx_hbm.at[pl.ds(cbase, chunks_per_w)], idx_v)
        pltpu.sync_copy(dst_hbm.at[pl.ds(cbase, chunks_per_w)], dst_v)

        def fire_group(g, sb):
            for i in range(grp):
                pltpu.async_copy(
                    table_hbm.at[idx_v.at[g * grp + i]], rows_v.at[sb + i], gsem
                )

        def drain(sem, n):
            for _ in range(n):
                pltpu.make_async_copy(
                    table_hbm.at[pl.ds(0, chunk)], rows_v.at[0], sem
                ).wait()

        fire_group(0, 0)

        def body(g, carry):
            sb = (g % 2) * grp
            nsb = grp - sb
            drain(gsem, grp)

            @pl.when(g >= 1)
            def _():
                drain(osem, grp)

            @pl.when(g + 1 < n_groups)
            def _():
                fire_group(g + 1, nsb)

            for i in range(grp):
                pltpu.async_copy(
                    rows_v.at[sb + i],
                    out_hbm.at[dst_v.at[g * grp + i]],
                    osem,
                )
            return carry

        lax.fori_loop(0, n_groups, body, 0)
        drain(osem, grp)

    return gather_k(idx2d, dst2d, table)


def _project(g2, W, b, bsz, npair):
    bm = 2048
    nb = bsz // bm

    def mm_k(x_ref, w_ref, b_ref, o_ref):
        t = pl.program_id(1)

        @pl.when(t == 0)
        def _():
            o_ref[...] = jnp.broadcast_to(b_ref[...], o_ref.shape)

        o_ref[...] += jnp.dot(
            x_ref[...], w_ref[...], preferred_element_type=jnp.float32
        )

    return pl.pallas_call(
        mm_k,
        grid=(nb, npair),
        in_specs=[
            pl.BlockSpec((bm, 2 * HIDDEN), lambda i, t: (t * nb + i, 0)),
            pl.BlockSpec((2 * HIDDEN, HIDDEN), lambda i, t: (t, 0)),
            pl.BlockSpec((1, HIDDEN), lambda i, t: (0, 0)),
        ],
        out_specs=pl.BlockSpec((bm, HIDDEN), lambda i, t: (i, 0)),
        out_shape=jax.ShapeDtypeStruct((bsz, HIDDEN), jnp.float32),
    )(g2, W, b.reshape(1, HIDDEN))


def kernel(int_vals, table, W, b):
    bsz, s = int_vals.shape
    npair = s // 2
    packed = _pack_table(table.T)
    table_lin = packed.reshape(2 * packed.shape[0], HIDDEN)
    idx2d = _remap_indices(int_vals).reshape(-1, CHUNK)
    bb = jnp.arange(bsz, dtype=jnp.int32)[:, None]
    ss = jnp.arange(s, dtype=jnp.int32)[None, :]
    dst2d = (2 * ((ss // 2) * bsz + bb) + ss % 2).reshape(-1, CHUNK)
    gathered = _gather_rows(idx2d, dst2d, table_lin)
    g2 = gathered.reshape(npair * bsz, 2 * HIDDEN)
    return _project(g2, W, b, bsz, npair)
